# Initial kernel scaffold; baseline (speedup 1.0000x reference)
import jax
import jax.numpy as jnp
from jax.experimental import pallas as pl


def kernel(pix_coord, coefficients, bias):
    raise NotImplementedError("write your pallas kernel here")


# speedup vs baseline: 38.4899x; 38.4899x over previous
import functools

import jax
import jax.numpy as jnp
from jax.experimental import pallas as pl

_H = 1000
_W = 2000
_PS = 20
_PCOLS = _W // _PS
_NT = 5


def _strip_kernel(x_ref, y_ref, w_ref, e_ref, o_ref):
    w = w_ref[0]
    ew = jax.lax.dot_general(
        w, e_ref[...], (((1,), (0,)), ((), ())),
        preferred_element_type=jnp.float32)

    def row(k):
        return jax.lax.slice_in_dim(ew, k, k + 1, axis=0)

    x = x_ref[0]
    y = y_ref[0]
    for c in range(3):
        base = c * 11
        px = row(base + 4)
        for t in (3, 2, 1, 0):
            px = px * x + row(base + t)
        py = row(base + 9)
        for t in (8, 7, 6, 5):
            py = py * y + row(base + t)
        o_ref[c, 0] = px + py + row(base + 10)


@functools.partial(jax.jit, static_argnums=())
def kernel(pix_coord, coefficients, bias):
    x = pix_coord[:, 0].reshape(_H // _PS, _PS, _W)
    y = pix_coord[:, 1].reshape(_H // _PS, _PS, _W)
    w = jnp.concatenate(
        [coefficients[..., 0], coefficients[..., 1], bias[..., None]],
        axis=-1)
    w = w.reshape(_H // _PS, _PCOLS, 33).transpose(0, 2, 1)
    e = (jnp.arange(_W, dtype=jnp.int32)[None, :] // _PS ==
         jnp.arange(_PCOLS, dtype=jnp.int32)[:, None]).astype(jnp.float32)

    grid = _H // _PS
    out = pl.pallas_call(
        _strip_kernel,
        grid=(grid,),
        in_specs=[
            pl.BlockSpec((1, _PS, _W), lambda i: (i, 0, 0)),
            pl.BlockSpec((1, _PS, _W), lambda i: (i, 0, 0)),
            pl.BlockSpec((1, 33, _PCOLS), lambda i: (i, 0, 0)),
            pl.BlockSpec((_PCOLS, _W), lambda i: (0, 0)),
        ],
        out_specs=pl.BlockSpec((3, 1, _PS, _W), lambda i: (0, i, 0, 0)),
        out_shape=jax.ShapeDtypeStruct((3, _H // _PS, _PS, _W), jnp.float32),
    )(x, y, w, e)
    return out.reshape(3, _H * _W)


# speedup vs baseline: 41.6647x; 1.0825x over previous
import functools

import jax
import jax.numpy as jnp
from jax.experimental import pallas as pl

_H = 1000
_W = 2000
_PS = 20
_PCOLS = _W // _PS
_NT = 5


def _strip_kernel(x_ref, y_ref, w_ref, e_ref, o_ref):
    w = w_ref[0]
    ew = jax.lax.dot_general(
        w, e_ref[...], (((1,), (0,)), ((), ())),
        preferred_element_type=jnp.float32)

    def row(k):
        return jax.lax.slice_in_dim(ew, k, k + 1, axis=0)

    x = x_ref[0, 0]
    y = y_ref[0, 0]
    for c in range(3):
        base = c * 11
        px = row(base + 4)
        for t in (3, 2, 1, 0):
            px = px * x + row(base + t)
        py = row(base + 9)
        for t in (8, 7, 6, 5):
            py = py * y + row(base + t)
        o_ref[c, 0] = px + py + row(base + 10)


@functools.partial(jax.jit, static_argnums=())
def kernel(pix_coord, coefficients, bias):
    xyt = pix_coord.T.reshape(2, _H // _PS, _PS, _W)
    w = jnp.concatenate(
        [coefficients[..., 0], coefficients[..., 1], bias[..., None]],
        axis=-1)
    w = w.reshape(_H // _PS, _PCOLS, 33).transpose(0, 2, 1)
    e = (jnp.arange(_W, dtype=jnp.int32)[None, :] // _PS ==
         jnp.arange(_PCOLS, dtype=jnp.int32)[:, None]).astype(jnp.float32)

    grid = _H // _PS
    out = pl.pallas_call(
        _strip_kernel,
        grid=(grid,),
        in_specs=[
            pl.BlockSpec((1, 1, _PS, _W), lambda i: (0, i, 0, 0)),
            pl.BlockSpec((1, 1, _PS, _W), lambda i: (1, i, 0, 0)),
            pl.BlockSpec((1, 33, _PCOLS), lambda i: (i, 0, 0)),
            pl.BlockSpec((_PCOLS, _W), lambda i: (0, 0)),
        ],
        out_specs=pl.BlockSpec((3, 1, _PS, _W), lambda i: (0, i, 0, 0)),
        out_shape=jax.ShapeDtypeStruct((3, _H // _PS, _PS, _W), jnp.float32),
    )(xyt, xyt, w, e)
    return out.reshape(3, _H * _W)


# speedup vs baseline: 41.7394x; 1.0018x over previous
import functools

import jax
import jax.numpy as jnp
from jax.experimental import pallas as pl

_H = 1000
_W = 2000
_PS = 20
_PCOLS = _W // _PS
_NT = 5


def _strip_kernel(x_ref, y_ref, w_ref, o_ref):
    w = w_ref[0]
    lane = jax.lax.broadcasted_iota(jnp.int32, (_PCOLS, _W), 1)
    sub = jax.lax.broadcasted_iota(jnp.int32, (_PCOLS, _W), 0)
    e = (lane // _PS == sub).astype(jnp.float32)
    ew = jax.lax.dot_general(
        w, e, (((1,), (0,)), ((), ())),
        preferred_element_type=jnp.float32)

    def row(k):
        return jax.lax.slice_in_dim(ew, k, k + 1, axis=0)

    x = x_ref[0, 0]
    y = y_ref[0, 0]
    for c in range(3):
        base = c * 11
        px = row(base + 4)
        for t in (3, 2, 1, 0):
            px = px * x + row(base + t)
        py = row(base + 9)
        for t in (8, 7, 6, 5):
            py = py * y + row(base + t)
        o_ref[c, 0] = px + py + row(base + 10)


@functools.partial(jax.jit, static_argnums=())
def kernel(pix_coord, coefficients, bias):
    xyt = pix_coord.T.reshape(2, _H // _PS, _PS, _W)
    w = jnp.concatenate(
        [coefficients[..., 0], coefficients[..., 1], bias[..., None]],
        axis=-1)
    w = w.reshape(_H // _PS, _PCOLS, 33).transpose(0, 2, 1)

    grid = _H // _PS
    out = pl.pallas_call(
        _strip_kernel,
        grid=(grid,),
        in_specs=[
            pl.BlockSpec((1, 1, _PS, _W), lambda i: (0, i, 0, 0)),
            pl.BlockSpec((1, 1, _PS, _W), lambda i: (1, i, 0, 0)),
            pl.BlockSpec((1, 33, _PCOLS), lambda i: (i, 0, 0)),
        ],
        out_specs=pl.BlockSpec((3, 1, _PS, _W), lambda i: (0, i, 0, 0)),
        out_shape=jax.ShapeDtypeStruct((3, _H // _PS, _PS, _W), jnp.float32),
    )(xyt, xyt, w)
    return out.reshape(3, _H * _W)
